# baseline (speedup 1.0000x reference)
import functools

import jax
import jax.numpy as jnp
from jax import lax
from jax.experimental import pallas as pl
from jax.experimental.pallas import tpu as pltpu
from jax.experimental.pallas import tpu_sc as plsc

D_MODEL = 1024
B, T = 4, 2048
ROWS = B * (T + 1)
NW = 32
ROWS_PER_W = (ROWS // NW) // 16 * 16
NBLK = ROWS_PER_W // 16
TAIL = ROWS - NW * ROWS_PER_W
IDS_PAD = NW * ROWS_PER_W + 16
BLK = 16 * D_MODEL


def _body(ids_hbm, out_hbm, ids_v, tail_v, buf0, buf1, sem0, sem1):
    nc = plsc.get_sparse_core_info().num_cores
    wid = lax.axis_index("s") * nc + lax.axis_index("c")
    base = wid * ROWS_PER_W

    pltpu.sync_copy(ids_hbm.at[pl.ds(base, ROWS_PER_W)], ids_v)
    pltpu.sync_copy(ids_hbm.at[pl.ds(NW * ROWS_PER_W, 16)], tail_v)

    zeros16 = jnp.zeros((16,), jnp.float32)
    ones16 = jnp.ones((16,), jnp.float32)
    iota16 = lax.iota(jnp.int32, 16)
    row_off = iota16 * D_MODEL

    def _zinit(i, c):
        col = i * 64
        for bref in (buf0, buf1):
            for off in range(0, 64, 16):
                bref[pl.ds(col + off, 16)] = zeros16
        return c

    lax.fori_loop(0, BLK // 64, _zinit, 0)

    bufs = (buf0, buf1)
    sems = (sem0, sem1)
    pend = [None, None]
    for k in range(NBLK):
        s = k % 2
        if pend[s] is not None:
            old_idx, dma = pend[s]
            dma.wait()
            plsc.store_scatter(bufs[s], [old_idx], zeros16)
        idx = row_off + ids_v[pl.ds(k * 16, 16)]
        plsc.store_scatter(bufs[s], [idx], ones16)
        dma = pltpu.async_copy(
            bufs[s], out_hbm.at[pl.ds((base + k * 16) * D_MODEL, BLK)],
            sems[s])
        pend[s] = (idx, dma)

    pend[0][1].wait()
    pend[1][1].wait()

    @pl.when(wid < TAIL)
    def _tail():
        plsc.store_scatter(buf0, [pend[0][0]], zeros16)
        tid = tail_v[pl.ds(0, 16)]
        lane = jnp.equal(iota16, wid)
        plsc.store_scatter(buf0, [tid], ones16, mask=lane)
        pltpu.sync_copy(
            buf0.at[pl.ds(0, D_MODEL)],
            out_hbm.at[pl.ds((NW * ROWS_PER_W + wid) * D_MODEL, D_MODEL)])


@functools.partial(jax.jit, static_argnums=())
def kernel(input_ids):
    padded = jnp.pad(input_ids.astype(jnp.int32), ((0, 0), (1, 0)),
                     mode="constant", constant_values=0)
    ids = jnp.pad(padded.reshape(ROWS), (0, IDS_PAD - ROWS))
    k = pl.kernel(
        _body,
        out_type=jax.ShapeDtypeStruct((ROWS * D_MODEL,), jnp.float32),
        mesh=plsc.VectorSubcoreMesh(core_axis_name="c", subcore_axis_name="s"),
        compiler_params=pltpu.CompilerParams(needs_layout_passes=False),
        scratch_types=[
            pltpu.VMEM((ROWS_PER_W,), jnp.int32),
            pltpu.VMEM((16,), jnp.int32),
            pltpu.VMEM((BLK,), jnp.float32),
            pltpu.VMEM((BLK,), jnp.float32),
            pltpu.SemaphoreType.DMA,
            pltpu.SemaphoreType.DMA,
        ],
    )
    out = k(ids)
    return out.reshape(B, T + 1, D_MODEL)


# speedup vs baseline: 1.1009x; 1.1009x over previous
import functools

import jax
import jax.numpy as jnp
from jax import lax
from jax.experimental import pallas as pl
from jax.experimental.pallas import tpu as pltpu
from jax.experimental.pallas import tpu_sc as plsc

D_MODEL = 1024
B, T = 4, 2048
NW = 32
WPB = NW // B
ROWS_PER_W = T // WPB
NBLK = ROWS_PER_W // 16
IDS_PAD = NW * ROWS_PER_W + 16


def _body(ids_hbm, out_hbm, ids_v, tail_v, buf0, buf1, sem0, sem1):
    nc = plsc.get_sparse_core_info().num_cores
    wid = lax.axis_index("s") * nc + lax.axis_index("c")
    b = wid // WPB
    base_t = (wid % WPB) * ROWS_PER_W

    pltpu.sync_copy(ids_hbm.at[pl.ds(wid * ROWS_PER_W, ROWS_PER_W)], ids_v)
    pltpu.sync_copy(ids_hbm.at[pl.ds(NW * ROWS_PER_W, 16)], tail_v)

    zeros16 = jnp.zeros((16,), jnp.float32)
    ones16 = jnp.ones((16,), jnp.float32)
    iota16 = lax.iota(jnp.int32, 16)

    def _zinit(i, c):
        r = i >> 4
        col = (i & 15) * 64
        for bref in (buf0, buf1):
            for off in range(0, 64, 16):
                bref[r, pl.ds(col + off, 16)] = zeros16
        return c

    lax.fori_loop(0, 256, _zinit, 0)

    bufs = (buf0, buf1)
    sems = (sem0, sem1)
    pend = [None, None]
    for k in range(NBLK):
        s = k % 2
        if pend[s] is not None:
            old_ids, dma = pend[s]
            dma.wait()
            plsc.store_scatter(bufs[s], [iota16, old_ids], zeros16)
        idv = ids_v[pl.ds(k * 16, 16)]
        plsc.store_scatter(bufs[s], [iota16, idv], ones16)
        dma = pltpu.async_copy(
            bufs[s], out_hbm.at[b, pl.ds(base_t + k * 16, 16), :], sems[s])
        pend[s] = (idv, dma)

    pend[0][1].wait()
    pend[1][1].wait()

    @pl.when(base_t == 0)
    def _tail():
        plsc.store_scatter(buf0, [iota16, pend[0][0]], zeros16)
        tid = tail_v[pl.ds(0, 16)]
        lane = jnp.equal(iota16, b)
        plsc.store_scatter(buf0, [jnp.zeros((16,), jnp.int32), tid],
                           ones16, mask=lane)
        pltpu.sync_copy(buf0.at[pl.ds(0, 1)],
                        out_hbm.at[b, pl.ds(T, 1), :])


@functools.partial(jax.jit, static_argnums=())
def kernel(input_ids):
    ids32 = input_ids.astype(jnp.int32)
    padded = jnp.pad(ids32, ((0, 0), (1, 0)), constant_values=0)
    ids = jnp.concatenate([
        padded[:, :T].reshape(NW * ROWS_PER_W),
        jnp.pad(padded[:, T], (0, 12)),
    ])
    k = pl.kernel(
        _body,
        out_type=jax.ShapeDtypeStruct((B, T + 1, D_MODEL), jnp.float32),
        mesh=plsc.VectorSubcoreMesh(core_axis_name="c", subcore_axis_name="s"),
        compiler_params=pltpu.CompilerParams(needs_layout_passes=False),
        scratch_types=[
            pltpu.VMEM((ROWS_PER_W,), jnp.int32),
            pltpu.VMEM((16,), jnp.int32),
            pltpu.VMEM((16, D_MODEL), jnp.float32),
            pltpu.VMEM((16, D_MODEL), jnp.float32),
            pltpu.SemaphoreType.DMA,
            pltpu.SemaphoreType.DMA,
        ],
    )
    return k(ids)
